# baseline (speedup 1.0000x reference)
import jax
import jax.numpy as jnp
from jax.experimental import pallas as pl
from jax.experimental.pallas import tpu as pltpu

_TOKENS = 16384
_DIM = 2048
_EXPERTS = 64
_HBLK = 512


def _softmax(logits):
    m = jnp.max(logits, axis=-1, keepdims=True)
    e = jnp.exp(logits - m)
    return e / jnp.sum(e, axis=-1, keepdims=True)


def _gate_block(xa_ref, xb_ref, w_ref, b_ref, o_ref):
    w = w_ref[...]
    b = b_ref[...]
    la = jnp.dot(xa_ref[...], w, preferred_element_type=jnp.float32) + b
    lb = jnp.dot(xb_ref[...], w, preferred_element_type=jnp.float32) + b
    o_ref[0:_HBLK, :] = _softmax(la)
    o_ref[_HBLK:, :] = _softmax(lb)


def kernel(x, W, b):
    b2 = b.reshape(1, _EXPERTS)
    grid = (_TOKENS // (2 * _HBLK),)
    return pl.pallas_call(
        _gate_block,
        grid=grid,
        in_specs=[
            pl.BlockSpec((_HBLK, _DIM), lambda i: (2 * i, 0)),
            pl.BlockSpec((_HBLK, _DIM), lambda i: (2 * i + 1, 0)),
            pl.BlockSpec((_DIM, _EXPERTS), lambda i: (0, 0)),
            pl.BlockSpec((1, _EXPERTS), lambda i: (0, 0)),
        ],
        out_specs=pl.BlockSpec((2 * _HBLK, _EXPERTS), lambda i: (i, 0)),
        out_shape=jax.ShapeDtypeStruct((_TOKENS, _EXPERTS), jnp.float32),
        compiler_params=pltpu.CompilerParams(
            dimension_semantics=("arbitrary",),
        ),
    )(x, x, W, b2)
